# Initial kernel scaffold; baseline (speedup 1.0000x reference)
import jax
import jax.numpy as jnp
from jax.experimental import pallas as pl


def kernel(x, edge_index, W1, b1, W2, b2, W_lin, b_lin):
    raise NotImplementedError("write your pallas kernel here")


# speedup vs baseline: 9.5338x; 9.5338x over previous
import functools

import jax
import jax.numpy as jnp
from jax import lax
from jax.experimental import pallas as pl
from jax.experimental.pallas import tpu as pltpu
from jax.experimental.pallas import tpu_sc as plsc

N = 10000
E = 320000
D = 128
OUT = 64

NC = 2
NS = 16
NW = NC * NS
CHUNK = 128
CH = 80
EPW = CH * CHUNK
EPAD = NW * EPW
TRASH = N
AGG_ROWS = 10240
DEG_ROWS = 10240
RB = 1000


_sc_mesh = plsc.VectorSubcoreMesh(core_axis_name="c", subcore_axis_name="s")


@functools.partial(
    pl.kernel,
    out_type=jax.ShapeDtypeStruct((NC, DEG_ROWS), jnp.float32),
    mesh=_sc_mesh,
    scratch_types=[
        pltpu.VMEM((CH, CHUNK), jnp.int32),
        pltpu.VMEM((CHUNK,), jnp.float32),
        pltpu.VMEM((DEG_ROWS // NS,), jnp.float32),
        pltpu.VMEM_SHARED((DEG_ROWS,), jnp.float32),
    ],
)
def _deg_kernel(dst_hbm, out_hbm, dst_v, ones_v, zbuf_v, hist_sp):
    c = lax.axis_index("c")
    s = lax.axis_index("s")
    wid = s * NC + c
    per = DEG_ROWS // NS

    zvec = jnp.zeros((16,), jnp.float32)
    for j in range(per // 16):
        zbuf_v[pl.ds(j * 16, 16)] = zvec
    ovec = jnp.ones((16,), jnp.float32)
    for j in range(CHUNK // 16):
        ones_v[pl.ds(j * 16, 16)] = ovec

    pltpu.sync_copy(zbuf_v, hist_sp.at[pl.ds(s * per, per)])
    pltpu.sync_copy(dst_hbm.at[wid], dst_v)
    plsc.subcore_barrier()

    @pl.loop(0, CH)
    def _(ch):
        pltpu.sync_copy(ones_v, hist_sp.at[dst_v.at[ch]], add=True)

    plsc.subcore_barrier()
    pltpu.sync_copy(hist_sp.at[pl.ds(s * per, per)],
                    out_hbm.at[c, pl.ds(s * per, per)])


@functools.partial(
    pl.kernel,
    out_type=jax.ShapeDtypeStruct((NC, AGG_ROWS, D), jnp.float32),
    mesh=_sc_mesh,
    scratch_types=[
        pltpu.VMEM((CH // 2, CHUNK), jnp.int32),
        pltpu.VMEM((CH // 2, CHUNK), jnp.int32),
        pltpu.VMEM((2, CHUNK, D), jnp.float32),
        pltpu.VMEM_SHARED((AGG_ROWS, D), jnp.float32),
        pltpu.SemaphoreType.DMA((2,)),
    ],
)
def _agg_kernel(g_hbm, src_hbm, dst_hbm, zero_hbm, out_hbm,
                src_v, dst_v, rows_v, agg_sp, sem):
    c = lax.axis_index("c")
    s = lax.axis_index("s")
    wid = s * NC + c
    per = AGG_ROWS // NS
    half = CH // 2

    pltpu.sync_copy(zero_hbm.at[pl.ds(s * per, per)],
                    agg_sp.at[pl.ds(s * per, per)])
    plsc.subcore_barrier()

    for phase in range(2):
        pltpu.sync_copy(src_hbm.at[wid, pl.ds(phase * half, half)], src_v)
        pltpu.sync_copy(dst_hbm.at[wid, pl.ds(phase * half, half)], dst_v)

        pltpu.async_copy(g_hbm.at[src_v.at[0]], rows_v.at[0], sem.at[0])
        pltpu.async_copy(g_hbm.at[src_v.at[1]], rows_v.at[1], sem.at[1])

        @pl.loop(0, half // 2)
        def _(g):
            for b in range(2):
                ch = g * 2 + b
                pltpu.make_async_copy(g_hbm.at[src_v.at[ch]], rows_v.at[b],
                                      sem.at[b]).wait()
                pltpu.sync_copy(rows_v.at[b], agg_sp.at[dst_v.at[ch]],
                                add=True)

                @pl.when(ch + 2 < half)
                def _():
                    pltpu.async_copy(g_hbm.at[src_v.at[ch + 2]],
                                     rows_v.at[b], sem.at[b])

    plsc.subcore_barrier()
    pltpu.sync_copy(agg_sp.at[pl.ds(s * per, per)],
                    out_hbm.at[c, pl.ds(s * per, per)])


def _g1_body(x_ref, w1_ref, h0_ref, h1_ref, g1_ref, dinv_ref):
    deg = h0_ref[...] + h1_ref[...] + 1.0
    dinv = lax.rsqrt(deg)
    h = jnp.dot(x_ref[...], w1_ref[...], preferred_element_type=jnp.float32)
    g1_ref[...] = h * dinv
    dinv_ref[...] = dinv


def _g2_body(p0_ref, p1_ref, g1_ref, dinv_ref, b1_ref, w2_ref, g2_ref):
    dinv = dinv_ref[...]
    u = jax.nn.relu((p0_ref[...] + p1_ref[...] + g1_ref[...]) * dinv
                    + b1_ref[...])
    g2_ref[...] = jnp.dot(u, w2_ref[...],
                          preferred_element_type=jnp.float32) * dinv


def _fin_body(q0_ref, q1_ref, g2_ref, dinv_ref, b2_ref, wl_ref, bl_ref,
              out_ref, emb_ref):
    emb = jax.nn.relu((q0_ref[...] + q1_ref[...] + g2_ref[...])
                      * dinv_ref[...] + b2_ref[...])
    emb_ref[...] = emb
    out_ref[...] = jnp.dot(emb, wl_ref[...],
                           preferred_element_type=jnp.float32) + bl_ref[...]


def _row_spec(cols):
    return pl.BlockSpec((RB, cols), lambda i: (i, 0))


def _full_spec(shape):
    return pl.BlockSpec(shape, lambda i: tuple(0 for _ in shape))


def kernel(x, edge_index, W1, b1, W2, b2, W_lin, b_lin):
    src = edge_index[0].astype(jnp.int32)
    dst = edge_index[1].astype(jnp.int32)
    pad = EPAD - E
    src_p = jnp.concatenate([src, jnp.zeros((pad,), jnp.int32)])
    dst_p = jnp.concatenate([dst, jnp.full((pad,), TRASH, jnp.int32)])
    src_p = src_p.reshape(NW, CH, CHUNK)
    dst_p = dst_p.reshape(NW, CH, CHUNK)
    zeros_agg = jnp.zeros((AGG_ROWS, D), jnp.float32)

    hist = _deg_kernel(dst_p)
    h0 = hist[0, :N].reshape(N, 1)
    h1 = hist[1, :N].reshape(N, 1)

    grid = (N // RB,)
    g1, dinv = pl.pallas_call(
        _g1_body,
        grid=grid,
        in_specs=[_row_spec(D), _full_spec((D, D)), _row_spec(1), _row_spec(1)],
        out_specs=[_row_spec(D), _row_spec(1)],
        out_shape=[jax.ShapeDtypeStruct((N, D), jnp.float32),
                   jax.ShapeDtypeStruct((N, 1), jnp.float32)],
    )(x, W1, h0, h1)

    agg1 = _agg_kernel(g1, src_p, dst_p, zeros_agg)[:, :N]

    g2 = pl.pallas_call(
        _g2_body,
        grid=grid,
        in_specs=[_row_spec(D), _row_spec(D), _row_spec(D), _row_spec(1),
                  _full_spec((D,)), _full_spec((D, D))],
        out_specs=_row_spec(D),
        out_shape=jax.ShapeDtypeStruct((N, D), jnp.float32),
    )(agg1[0], agg1[1], g1, dinv, b1, W2)

    agg2 = _agg_kernel(g2, src_p, dst_p, zeros_agg)[:, :N]

    out, emb = pl.pallas_call(
        _fin_body,
        grid=grid,
        in_specs=[_row_spec(D), _row_spec(D), _row_spec(D), _row_spec(1),
                  _full_spec((D,)), _full_spec((D, OUT)), _full_spec((OUT,))],
        out_specs=[_row_spec(OUT), _row_spec(D)],
        out_shape=[jax.ShapeDtypeStruct((N, OUT), jnp.float32),
                   jax.ShapeDtypeStruct((N, D), jnp.float32)],
    )(agg2[0], agg2[1], g2, dinv, b2, W_lin, b_lin)

    return (out, emb)
